# baseline (device time: 194141 ns/iter reference)
import jax
import jax.numpy as jnp
from jax import lax
from jax.experimental import pallas as pl
from jax.experimental.pallas import tpu as pltpu

N_DEV = 4


def kernel(Q, K, V):
    b, s, nh, d = Q.shape
    scale = d ** -0.5
    w = nh * d

    Qt = (Q.reshape(s, w) * scale).astype(jnp.bfloat16)
    Kt = K.reshape(s, w).astype(jnp.bfloat16)
    Vt = V.reshape(s, w).astype(jnp.bfloat16)

    def body(q_ref, k_ref, v_ref, o_ref, kv_ref, l_ref,
             send_sems, recv_sems):
        c = pl.program_id(0)
        hh = pl.program_id(1)
        my = lax.axis_index("i")
        left = lax.rem(my + N_DEV - 1, N_DEV)
        right = lax.rem(my + 1, N_DEV)

        segs = ((0, 4, True), (4, 8, True), (8, 12, False), (12, 16, False))

        def seg_rdmas(i, j):
            s0, s1, rightward = segs[j]
            dev = right if rightward else left
            c0, c1 = s0 * d, s1 * d
            if i == 0:
                src_k = k_ref.at[:, c0:c1]
                src_v = v_ref.at[:, c0:c1]
            else:
                src_k = kv_ref.at[i - 1, 0, :, c0:c1]
                src_v = kv_ref.at[i - 1, 1, :, c0:c1]
            mk = pltpu.make_async_remote_copy
            return [
                mk(src_ref=src_k, dst_ref=kv_ref.at[i, 0, :, c0:c1],
                   send_sem=send_sems.at[i, 2 * j],
                   recv_sem=recv_sems.at[i, 2 * j],
                   device_id=(dev,), device_id_type=pl.DeviceIdType.MESH),
                mk(src_ref=src_v, dst_ref=kv_ref.at[i, 1, :, c0:c1],
                   send_sem=send_sems.at[i, 2 * j + 1],
                   recv_sem=recv_sems.at[i, 2 * j + 1],
                   device_id=(dev,), device_id_type=pl.DeviceIdType.MESH),
            ]

        @pl.when(jnp.logical_and(c == 0, hh == 0))
        def _():
            barrier_sem = pltpu.get_barrier_semaphore()
            for nbr in (left, right):
                pl.semaphore_signal(
                    barrier_sem, inc=1,
                    device_id=(nbr,), device_id_type=pl.DeviceIdType.MESH,
                )
            pl.semaphore_wait(barrier_sem, 2)
            for j in range(4):
                for r in seg_rdmas(0, j):
                    r.start()

        for i in (1, 2, 3):
            for j in range(4):
                @pl.when(jnp.logical_and(c == i, hh == segs[j][0]))
                def _(i=i, j=j):
                    for r in seg_rdmas(i - 1, j):
                        r.wait()
                    if i < 3:
                        for r in seg_rdmas(i, j):
                            r.start()

        cm1 = jnp.maximum(c - 1, 0)
        col = hh * d
        q_h = q_ref[:, pl.ds(col, d)]

        def pv_ext_of(k_c, v_c):
            s_c = lax.dot_general(
                q_h, k_c, (((1,), (1,)), ((), ())),
                preferred_element_type=jnp.float32,
            )
            p = jnp.exp(s_c.astype(jnp.bfloat16))
            v_ext = jnp.concatenate(
                [v_c, jnp.ones((s, d), dtype=jnp.bfloat16)], axis=1
            )
            return lax.dot_general(
                p, v_ext, (((1,), (0,)), ((), ())),
                preferred_element_type=jnp.float32,
            )

        @pl.when(c == 0)
        def _():
            pv_ext = pv_ext_of(
                k_ref[:, pl.ds(col, d)], v_ref[:, pl.ds(col, d)]
            )
            o_ref[:, pl.ds(col, d)] = pv_ext[:, :d]
            l_ref[:, pl.ds(col, d)] = pv_ext[:, d:]

        @pl.when(jnp.logical_and(c > 0, c < N_DEV - 1))
        def _():
            pv_ext = pv_ext_of(
                kv_ref[cm1, 0, :, pl.ds(col, d)],
                kv_ref[cm1, 1, :, pl.ds(col, d)],
            )
            o_ref[:, pl.ds(col, d)] += pv_ext[:, :d]
            l_ref[:, pl.ds(col, d)] += pv_ext[:, d:]

        @pl.when(c == N_DEV - 1)
        def _():
            pv_ext = pv_ext_of(
                kv_ref[cm1, 0, :, pl.ds(col, d)],
                kv_ref[cm1, 1, :, pl.ds(col, d)],
            )
            l_new = l_ref[:, pl.ds(col, d)] + pv_ext[:, d:]
            o_ref[:, pl.ds(col, d)] = (
                o_ref[:, pl.ds(col, d)] + pv_ext[:, :d]
            ) / l_new

    out = pl.pallas_call(
        body,
        grid=(N_DEV, nh),
        out_shape=jax.ShapeDtypeStruct((s, w), jnp.float32),
        in_specs=[
            pl.BlockSpec(memory_space=pltpu.VMEM),
            pl.BlockSpec(memory_space=pltpu.VMEM),
            pl.BlockSpec(memory_space=pltpu.VMEM),
        ],
        out_specs=pl.BlockSpec(memory_space=pltpu.VMEM),
        scratch_shapes=[
            pltpu.VMEM((N_DEV - 1, 2, s, w), jnp.bfloat16),
            pltpu.VMEM((s, w), jnp.float32),
            pltpu.SemaphoreType.DMA((N_DEV - 1, 8)),
            pltpu.SemaphoreType.DMA((N_DEV - 1, 8)),
        ],
        compiler_params=pltpu.CompilerParams(
            collective_id=0, vmem_limit_bytes=63 * 1024 * 1024
        ),
    )(Qt, Kt, Vt)
    return out.reshape(b, s, nh, d)
